# baseline (speedup 1.0000x reference)
import functools

import numpy as np

import jax
import jax.numpy as jnp
from jax.experimental import pallas as pl
from jax.experimental.pallas import tpu as pltpu


def _sel(dx_k, w_n, p_n, pj_n):
    s = np.zeros((dx_k, w_n, p_n, pj_n), np.float32)
    for dx in range(dx_k):
        for p in range(p_n):
            for pj in range(pj_n):
                s[dx, 2 * pj + p + dx, p, pj] = 1.0
    return s


_S1 = _sel(5, 32, 2, 14)
_S2 = _sel(5, 14, 2, 5)


def _lenet_kernel(xt_ref, w1_ref, b1_ref, w2_ref, b2_ref,
                  f1_ref, f1b_ref, f2_ref, f2b_ref, f3_ref, f3b_ref,
                  o_ref, *, tile_b):
    x = xt_ref[...]
    a1 = jnp.concatenate([x[kh:kh + 28] for kh in range(5)], axis=2)
    y1 = jnp.dot(a1.reshape(28 * tile_b, 480), w1_ref[...],
                 preferred_element_type=jnp.float32)
    y1 = y1.reshape(14, 2, tile_b, 256)
    h1 = jnp.maximum(
        jnp.maximum(jnp.maximum(y1[:, 0, :, :128], y1[:, 0, :, 128:]),
                    jnp.maximum(y1[:, 1, :, :128], y1[:, 1, :, 128:]))
        + b1_ref[...], 0.0)

    a2 = jnp.concatenate([h1[kh:kh + 10] for kh in range(5)], axis=2)
    y2 = jnp.dot(a2.reshape(10 * tile_b, 640), w2_ref[...],
                 preferred_element_type=jnp.float32)
    y2 = y2.reshape(5, 2, tile_b, 256)
    feat = jnp.maximum(
        jnp.maximum(jnp.maximum(y2[:, 0, :, :128], y2[:, 0, :, 128:]),
                    jnp.maximum(y2[:, 1, :, :128], y2[:, 1, :, 128:]))
        + b2_ref[...], 0.0)

    z = jnp.dot(feat[0], f1_ref[0],
                preferred_element_type=jnp.float32)
    for h in range(1, 5):
        z = z + jnp.dot(feat[h], f1_ref[h],
                        preferred_element_type=jnp.float32)
    z = jnp.maximum(z + f1b_ref[...], 0.0)
    z = jnp.dot(z, f2_ref[...], preferred_element_type=jnp.float32)
    z = jnp.maximum(z + f2b_ref[...], 0.0)
    o_ref[...] = jnp.dot(z, f3_ref[...],
                         preferred_element_type=jnp.float32) + f3b_ref[...]


def kernel(x, conv1_wcol, conv1_b, conv2_wcol, conv2_b,
           fc1_w, fc1_b, fc2_w, fc2_b, fc3_w, fc3_b):
    B = x.shape[0]
    xt = jnp.transpose(x.astype(jnp.bfloat16), (2, 0, 1, 3)).reshape(32, B, 96)

    w1 = conv1_wcol.reshape(5, 5, 8, 6)[:, :, :3, :]
    w1_wide = jnp.einsum('dwpj,kdcn->kcwpnj', _S1, w1).reshape(5, 96, 2, 84)
    w1_wide = jnp.pad(w1_wide, ((0, 0), (0, 0), (0, 0), (0, 44)))
    w1_wide = w1_wide.reshape(480, 256).astype(jnp.bfloat16)
    w2 = conv2_wcol.reshape(5, 5, 8, 16)[:, :, :6, :]
    w2_wide = jnp.einsum('dwpj,kdcn->kcwpnj', _S2, w2).reshape(5, 84, 2, 80)
    w2_wide = jnp.pad(w2_wide, ((0, 0), (0, 44), (0, 0), (0, 48)))
    w2_wide = w2_wide.reshape(640, 256)
    b1e = jnp.repeat(conv1_b.reshape(6, 1), 14, axis=1).reshape(1, 84)
    b1e = jnp.pad(b1e, ((0, 0), (0, 44)))
    b2e = jnp.repeat(conv2_b.reshape(16, 1), 5, axis=1).reshape(1, 80)
    b2e = jnp.pad(b2e, ((0, 0), (0, 48)))
    f1 = fc1_w.reshape(5, 5, 16, 120).transpose(0, 2, 1, 3).reshape(5, 80, 120)
    f1 = jnp.pad(f1, ((0, 0), (0, 48), (0, 0)))

    tile_b = min(256, B)
    Bp = (B + tile_b - 1) // tile_b * tile_b
    if Bp > B:
        xt = jnp.pad(xt, ((0, 0), (0, Bp - B), (0, 0)))

    n_out = fc3_w.shape[1]
    out = pl.pallas_call(
        functools.partial(_lenet_kernel, tile_b=tile_b),
        out_shape=jax.ShapeDtypeStruct((Bp, n_out), jnp.float32),
        grid=(Bp // tile_b,),
        in_specs=[
            pl.BlockSpec((32, tile_b, 96), lambda i: (0, i, 0)),
            pl.BlockSpec((480, 256), lambda i: (0, 0)),
            pl.BlockSpec((1, 128), lambda i: (0, 0)),
            pl.BlockSpec((640, 256), lambda i: (0, 0)),
            pl.BlockSpec((1, 128), lambda i: (0, 0)),
            pl.BlockSpec((5, 128, 120), lambda i: (0, 0, 0)),
            pl.BlockSpec((1, 120), lambda i: (0, 0)),
            pl.BlockSpec((120, 84), lambda i: (0, 0)),
            pl.BlockSpec((1, 84), lambda i: (0, 0)),
            pl.BlockSpec((84, 10), lambda i: (0, 0)),
            pl.BlockSpec((1, 10), lambda i: (0, 0)),
        ],
        out_specs=pl.BlockSpec((tile_b, n_out), lambda i: (i, 0)),
        compiler_params=pltpu.CompilerParams(
            dimension_semantics=("parallel",),
            vmem_limit_bytes=64 * 1024 * 1024),
        cost_estimate=pl.CostEstimate(
            flops=2 * Bp * (28 * 640 * 256 + 10 * 640 * 256 + 5 * 128 * 120
                            + 120 * 84 + 84 * 10),
            transcendentals=0,
            bytes_accessed=2 * Bp * 32 * 96 + 4 * Bp * n_out),
    )(xt, w1_wide, b1e, w2_wide, b2e,
      f1, fc1_b, fc2_w, fc2_b, fc3_w, fc3_b)
    return out[:B]


# speedup vs baseline: 1.0851x; 1.0851x over previous
import functools

import numpy as np

import jax
import jax.numpy as jnp
from jax.experimental import pallas as pl
from jax.experimental.pallas import tpu as pltpu


def _sel(dx_k, w_n, p_n, pj_n):
    s = np.zeros((dx_k, w_n, p_n, pj_n), np.float32)
    for dx in range(dx_k):
        for p in range(p_n):
            for pj in range(pj_n):
                s[dx, 2 * pj + p + dx, p, pj] = 1.0
    return s


_S1 = _sel(5, 32, 2, 14)
_S2 = _sel(5, 14, 2, 5)


def _lenet_kernel(xt_ref, w1_ref, b1_ref, w2_ref, b2_ref,
                  f1_ref, f1b_ref, f2_ref, f2b_ref, f3_ref, f3b_ref,
                  o_ref, *, tile_b):
    x = xt_ref[...]
    a1 = jnp.concatenate([x[kh:kh + 28] for kh in range(5)], axis=2)
    y1 = jnp.dot(a1.reshape(28 * tile_b, 480), w1_ref[...],
                 preferred_element_type=jnp.float32)
    y1 = y1.reshape(14, 2, tile_b, 256)
    h1 = jnp.maximum(
        jnp.maximum(jnp.maximum(y1[:, 0, :, :128], y1[:, 0, :, 128:]),
                    jnp.maximum(y1[:, 1, :, :128], y1[:, 1, :, 128:]))
        + b1_ref[...], 0.0)

    a2 = jnp.concatenate([h1[kh:kh + 10, :, :96] for kh in range(5)], axis=2)
    y2 = jnp.dot(a2.reshape(10 * tile_b, 480), w2_ref[...],
                 preferred_element_type=jnp.float32)
    y2 = y2.reshape(5, 2, tile_b, 256)
    feat = jnp.maximum(
        jnp.maximum(jnp.maximum(y2[:, 0, :, :128], y2[:, 0, :, 128:]),
                    jnp.maximum(y2[:, 1, :, :128], y2[:, 1, :, 128:]))
        + b2_ref[...], 0.0)

    z = jnp.dot(feat[0], f1_ref[0],
                preferred_element_type=jnp.float32)
    for h in range(1, 5):
        z = z + jnp.dot(feat[h], f1_ref[h],
                        preferred_element_type=jnp.float32)
    z = jnp.maximum(z + f1b_ref[...], 0.0)
    z = jnp.dot(z, f2_ref[...], preferred_element_type=jnp.float32)
    z = jnp.maximum(z + f2b_ref[...], 0.0)
    o_ref[...] = jnp.dot(z, f3_ref[...],
                         preferred_element_type=jnp.float32) + f3b_ref[...]


def kernel(x, conv1_wcol, conv1_b, conv2_wcol, conv2_b,
           fc1_w, fc1_b, fc2_w, fc2_b, fc3_w, fc3_b):
    B = x.shape[0]
    xt = jnp.transpose(x.astype(jnp.bfloat16), (2, 0, 1, 3)).reshape(32, B, 96)

    w1 = conv1_wcol.reshape(5, 5, 8, 6)[:, :, :3, :]
    w1_wide = jnp.einsum('dwpj,kdcn->kcwpnj', _S1, w1).reshape(5, 96, 2, 84)
    w1_wide = jnp.pad(w1_wide, ((0, 0), (0, 0), (0, 0), (0, 44)))
    w1_wide = w1_wide.reshape(480, 256).astype(jnp.bfloat16)
    w2 = conv2_wcol.reshape(5, 5, 8, 16)[:, :, :6, :]
    w2_wide = jnp.einsum('dwpj,kdcn->kcwpnj', _S2, w2).reshape(5, 84, 2, 80)
    w2_wide = jnp.pad(w2_wide, ((0, 0), (0, 12), (0, 0), (0, 48)))
    w2_wide = w2_wide.reshape(480, 256)
    b1e = jnp.repeat(conv1_b.reshape(6, 1), 14, axis=1).reshape(1, 84)
    b1e = jnp.pad(b1e, ((0, 0), (0, 44)))
    b2e = jnp.repeat(conv2_b.reshape(16, 1), 5, axis=1).reshape(1, 80)
    b2e = jnp.pad(b2e, ((0, 0), (0, 48)))
    f1 = fc1_w.reshape(5, 5, 16, 120).transpose(0, 2, 1, 3).reshape(5, 80, 120)
    f1 = jnp.pad(f1, ((0, 0), (0, 48), (0, 0)))

    tile_b = min(256, B)
    Bp = (B + tile_b - 1) // tile_b * tile_b
    if Bp > B:
        xt = jnp.pad(xt, ((0, 0), (0, Bp - B), (0, 0)))

    n_out = fc3_w.shape[1]
    out = pl.pallas_call(
        functools.partial(_lenet_kernel, tile_b=tile_b),
        out_shape=jax.ShapeDtypeStruct((Bp, n_out), jnp.float32),
        grid=(Bp // tile_b,),
        in_specs=[
            pl.BlockSpec((32, tile_b, 96), lambda i: (0, i, 0)),
            pl.BlockSpec((480, 256), lambda i: (0, 0)),
            pl.BlockSpec((1, 128), lambda i: (0, 0)),
            pl.BlockSpec((480, 256), lambda i: (0, 0)),
            pl.BlockSpec((1, 128), lambda i: (0, 0)),
            pl.BlockSpec((5, 128, 120), lambda i: (0, 0, 0)),
            pl.BlockSpec((1, 120), lambda i: (0, 0)),
            pl.BlockSpec((120, 84), lambda i: (0, 0)),
            pl.BlockSpec((1, 84), lambda i: (0, 0)),
            pl.BlockSpec((84, 10), lambda i: (0, 0)),
            pl.BlockSpec((1, 10), lambda i: (0, 0)),
        ],
        out_specs=pl.BlockSpec((tile_b, n_out), lambda i: (i, 0)),
        compiler_params=pltpu.CompilerParams(
            dimension_semantics=("parallel",),
            vmem_limit_bytes=64 * 1024 * 1024),
        cost_estimate=pl.CostEstimate(
            flops=2 * Bp * (28 * 640 * 256 + 10 * 640 * 256 + 5 * 128 * 120
                            + 120 * 84 + 84 * 10),
            transcendentals=0,
            bytes_accessed=2 * Bp * 32 * 96 + 4 * Bp * n_out),
    )(xt, w1_wide, b1e, w2_wide, b2e,
      f1, fc1_b, fc2_w, fc2_b, fc3_w, fc3_b)
    return out[:B]


# speedup vs baseline: 1.1021x; 1.0157x over previous
import functools

import numpy as np

import jax
import jax.numpy as jnp
from jax.experimental import pallas as pl
from jax.experimental.pallas import tpu as pltpu


def _sel(dx_k, w_n, p_n, pj_n):
    s = np.zeros((dx_k, w_n, p_n, pj_n), np.float32)
    for dx in range(dx_k):
        for p in range(p_n):
            for pj in range(pj_n):
                s[dx, 2 * pj + p + dx, p, pj] = 1.0
    return s


_S1 = _sel(5, 32, 2, 14)
_S2 = _sel(5, 14, 2, 5)


def _lenet_kernel(xt_ref, w1_ref, b1_ref, w2_ref, b2_ref,
                  f1_ref, f1b_ref, f2_ref, f2b_ref, f3_ref, f3b_ref,
                  o_ref, *, tile_b):
    x = xt_ref[...]
    a1 = jnp.concatenate([x[kh:kh + 28] for kh in range(5)], axis=2)
    y1 = jnp.dot(a1.reshape(28 * tile_b, 480), w1_ref[...],
                 preferred_element_type=jnp.float32)
    y1 = y1.reshape(14, 2, tile_b, 256)
    h1 = jnp.maximum(
        jnp.maximum(jnp.maximum(y1[:, 0, :, :128], y1[:, 0, :, 128:]),
                    jnp.maximum(y1[:, 1, :, :128], y1[:, 1, :, 128:]))
        + b1_ref[...], 0.0).astype(jnp.bfloat16)

    a2 = jnp.concatenate([h1[kh:kh + 10, :, :96] for kh in range(5)], axis=2)
    y2 = jnp.dot(a2.reshape(10 * tile_b, 480), w2_ref[...],
                 preferred_element_type=jnp.float32)
    y2 = y2.reshape(5, 2, tile_b, 256)
    feat = jnp.maximum(
        jnp.maximum(jnp.maximum(y2[:, 0, :, :128], y2[:, 0, :, 128:]),
                    jnp.maximum(y2[:, 1, :, :128], y2[:, 1, :, 128:]))
        + b2_ref[...], 0.0)

    z = jnp.dot(feat[0], f1_ref[0],
                preferred_element_type=jnp.float32)
    for h in range(1, 5):
        z = z + jnp.dot(feat[h], f1_ref[h],
                        preferred_element_type=jnp.float32)
    z = jnp.maximum(z + f1b_ref[...], 0.0)
    z = jnp.dot(z, f2_ref[...], preferred_element_type=jnp.float32)
    z = jnp.maximum(z + f2b_ref[...], 0.0)
    o_ref[...] = jnp.dot(z, f3_ref[...],
                         preferred_element_type=jnp.float32) + f3b_ref[...]


def kernel(x, conv1_wcol, conv1_b, conv2_wcol, conv2_b,
           fc1_w, fc1_b, fc2_w, fc2_b, fc3_w, fc3_b):
    B = x.shape[0]
    xt = jnp.transpose(x.astype(jnp.bfloat16), (2, 0, 1, 3)).reshape(32, B, 96)

    w1 = conv1_wcol.reshape(5, 5, 8, 6)[:, :, :3, :]
    w1_wide = jnp.einsum('dwpj,kdcn->kcwpnj', _S1, w1).reshape(5, 96, 2, 84)
    w1_wide = jnp.pad(w1_wide, ((0, 0), (0, 0), (0, 0), (0, 44)))
    w1_wide = w1_wide.reshape(480, 256).astype(jnp.bfloat16)
    w2 = conv2_wcol.reshape(5, 5, 8, 16)[:, :, :6, :]
    w2_wide = jnp.einsum('dwpj,kdcn->kcwpnj', _S2, w2).reshape(5, 84, 2, 80)
    w2_wide = jnp.pad(w2_wide, ((0, 0), (0, 12), (0, 0), (0, 48)))
    w2_wide = w2_wide.reshape(480, 256).astype(jnp.bfloat16)
    b1e = jnp.repeat(conv1_b.reshape(6, 1), 14, axis=1).reshape(1, 84)
    b1e = jnp.pad(b1e, ((0, 0), (0, 44)))
    b2e = jnp.repeat(conv2_b.reshape(16, 1), 5, axis=1).reshape(1, 80)
    b2e = jnp.pad(b2e, ((0, 0), (0, 48)))
    f1 = fc1_w.reshape(5, 5, 16, 120).transpose(0, 2, 1, 3).reshape(5, 80, 120)
    f1 = jnp.pad(f1, ((0, 0), (0, 48), (0, 0)))

    tile_b = min(256, B)
    Bp = (B + tile_b - 1) // tile_b * tile_b
    if Bp > B:
        xt = jnp.pad(xt, ((0, 0), (0, Bp - B), (0, 0)))

    n_out = fc3_w.shape[1]
    out = pl.pallas_call(
        functools.partial(_lenet_kernel, tile_b=tile_b),
        out_shape=jax.ShapeDtypeStruct((Bp, n_out), jnp.float32),
        grid=(Bp // tile_b,),
        in_specs=[
            pl.BlockSpec((32, tile_b, 96), lambda i: (0, i, 0)),
            pl.BlockSpec((480, 256), lambda i: (0, 0)),
            pl.BlockSpec((1, 128), lambda i: (0, 0)),
            pl.BlockSpec((480, 256), lambda i: (0, 0)),
            pl.BlockSpec((1, 128), lambda i: (0, 0)),
            pl.BlockSpec((5, 128, 120), lambda i: (0, 0, 0)),
            pl.BlockSpec((1, 120), lambda i: (0, 0)),
            pl.BlockSpec((120, 84), lambda i: (0, 0)),
            pl.BlockSpec((1, 84), lambda i: (0, 0)),
            pl.BlockSpec((84, 10), lambda i: (0, 0)),
            pl.BlockSpec((1, 10), lambda i: (0, 0)),
        ],
        out_specs=pl.BlockSpec((tile_b, n_out), lambda i: (i, 0)),
        compiler_params=pltpu.CompilerParams(
            dimension_semantics=("parallel",),
            vmem_limit_bytes=64 * 1024 * 1024),
        cost_estimate=pl.CostEstimate(
            flops=2 * Bp * (28 * 640 * 256 + 10 * 640 * 256 + 5 * 128 * 120
                            + 120 * 84 + 84 * 10),
            transcendentals=0,
            bytes_accessed=2 * Bp * 32 * 96 + 4 * Bp * n_out),
    )(xt, w1_wide, b1e, w2_wide, b2e,
      f1, fc1_b, fc2_w, fc2_b, fc3_w, fc3_b)
    return out[:B]


# speedup vs baseline: 1.1483x; 1.0419x over previous
import functools

import numpy as np

import jax
import jax.numpy as jnp
from jax.experimental import pallas as pl
from jax.experimental.pallas import tpu as pltpu


def _sel(dx_k, w_n, p_n, pj_n):
    s = np.zeros((dx_k, w_n, p_n, pj_n), np.float32)
    for dx in range(dx_k):
        for p in range(p_n):
            for pj in range(pj_n):
                s[dx, 2 * pj + p + dx, p, pj] = 1.0
    return s


_S1 = _sel(5, 32, 2, 14)
_S2 = _sel(5, 14, 2, 5)


def _lenet_kernel(xt_ref, w1_ref, b1_ref, w2_ref, b2_ref,
                  f1_ref, f1b_ref, f2_ref, f2b_ref, f3_ref, f3b_ref,
                  o_ref, *, tile_b):
    x = xt_ref[...]
    a1 = jnp.concatenate([x[kh:kh + 28] for kh in range(5)], axis=2)
    y1 = jnp.dot(a1.reshape(28 * tile_b, 480), w1_ref[...],
                 preferred_element_type=jnp.float32)
    y1 = y1.reshape(14, 2, tile_b, 256)
    h1 = jnp.maximum(
        jnp.maximum(jnp.maximum(y1[:, 0, :, :128], y1[:, 0, :, 128:]),
                    jnp.maximum(y1[:, 1, :, :128], y1[:, 1, :, 128:]))
        + b1_ref[...], 0.0).astype(jnp.bfloat16)

    a2 = jnp.concatenate([h1[kh:kh + 10, :, :96] for kh in range(5)], axis=2)
    y2 = jnp.dot(a2.reshape(10 * tile_b, 480), w2_ref[...],
                 preferred_element_type=jnp.float32)
    y2 = y2.reshape(5, 2, tile_b, 256)
    feat = jnp.maximum(
        jnp.maximum(jnp.maximum(y2[:, 0, :, :128], y2[:, 0, :, 128:]),
                    jnp.maximum(y2[:, 1, :, :128], y2[:, 1, :, 128:]))
        + b2_ref[...], 0.0)

    z = jnp.dot(feat[0], f1_ref[0],
                preferred_element_type=jnp.float32)
    for h in range(1, 5):
        z = z + jnp.dot(feat[h], f1_ref[h],
                        preferred_element_type=jnp.float32)
    z = jnp.maximum(z + f1b_ref[...], 0.0)
    z = jnp.dot(z, f2_ref[...], preferred_element_type=jnp.float32)
    z = jnp.maximum(z + f2b_ref[...], 0.0)
    o_ref[...] = jnp.dot(z, f3_ref[...],
                         preferred_element_type=jnp.float32) + f3b_ref[...]


def kernel(x, conv1_wcol, conv1_b, conv2_wcol, conv2_b,
           fc1_w, fc1_b, fc2_w, fc2_b, fc3_w, fc3_b):
    B = x.shape[0]
    xt = jnp.transpose(x.astype(jnp.bfloat16), (2, 0, 1, 3)).reshape(32, B, 96)

    w1 = conv1_wcol.reshape(5, 5, 8, 6)[:, :, :3, :]
    w1_wide = jnp.einsum('dwpj,kdcn->kcwpnj', _S1, w1).reshape(5, 96, 2, 84)
    w1_wide = jnp.pad(w1_wide, ((0, 0), (0, 0), (0, 0), (0, 44)))
    w1_wide = w1_wide.reshape(480, 256).astype(jnp.bfloat16)
    w2 = conv2_wcol.reshape(5, 5, 8, 16)[:, :, :6, :]
    w2_wide = jnp.einsum('dwpj,kdcn->kcwpnj', _S2, w2).reshape(5, 84, 2, 80)
    w2_wide = jnp.pad(w2_wide, ((0, 0), (0, 12), (0, 0), (0, 48)))
    w2_wide = w2_wide.reshape(480, 256).astype(jnp.bfloat16)
    b1e = jnp.repeat(conv1_b.reshape(6, 1), 14, axis=1).reshape(1, 84)
    b1e = jnp.pad(b1e, ((0, 0), (0, 44)))
    b2e = jnp.repeat(conv2_b.reshape(16, 1), 5, axis=1).reshape(1, 80)
    b2e = jnp.pad(b2e, ((0, 0), (0, 48)))
    f1 = fc1_w.reshape(5, 5, 16, 120).transpose(0, 2, 1, 3).reshape(5, 80, 120)
    f1 = jnp.pad(f1, ((0, 0), (0, 48), (0, 0)))

    tile_b = min(512, B)
    Bp = (B + tile_b - 1) // tile_b * tile_b
    if Bp > B:
        xt = jnp.pad(xt, ((0, 0), (0, Bp - B), (0, 0)))

    n_out = fc3_w.shape[1]
    out = pl.pallas_call(
        functools.partial(_lenet_kernel, tile_b=tile_b),
        out_shape=jax.ShapeDtypeStruct((Bp, n_out), jnp.float32),
        grid=(Bp // tile_b,),
        in_specs=[
            pl.BlockSpec((32, tile_b, 96), lambda i: (0, i, 0)),
            pl.BlockSpec((480, 256), lambda i: (0, 0)),
            pl.BlockSpec((1, 128), lambda i: (0, 0)),
            pl.BlockSpec((480, 256), lambda i: (0, 0)),
            pl.BlockSpec((1, 128), lambda i: (0, 0)),
            pl.BlockSpec((5, 128, 120), lambda i: (0, 0, 0)),
            pl.BlockSpec((1, 120), lambda i: (0, 0)),
            pl.BlockSpec((120, 84), lambda i: (0, 0)),
            pl.BlockSpec((1, 84), lambda i: (0, 0)),
            pl.BlockSpec((84, 10), lambda i: (0, 0)),
            pl.BlockSpec((1, 10), lambda i: (0, 0)),
        ],
        out_specs=pl.BlockSpec((tile_b, n_out), lambda i: (i, 0)),
        compiler_params=pltpu.CompilerParams(
            dimension_semantics=("parallel",),
            vmem_limit_bytes=64 * 1024 * 1024),
        cost_estimate=pl.CostEstimate(
            flops=2 * Bp * (28 * 640 * 256 + 10 * 640 * 256 + 5 * 128 * 120
                            + 120 * 84 + 84 * 10),
            transcendentals=0,
            bytes_accessed=2 * Bp * 32 * 96 + 4 * Bp * n_out),
    )(xt, w1_wide, b1e, w2_wide, b2e,
      f1, fc1_b, fc2_w, fc2_b, fc3_w, fc3_b)
    return out[:B]
